# baseline (speedup 1.0000x reference)
import functools

import jax
import jax.numpy as jnp
from jax import lax
from jax.experimental import pallas as pl
from jax.experimental.pallas import tpu as pltpu
from jax.experimental.pallas import tpu_sc as plsc

B = 16384
D = 256
NC = 2
NS = 16
NW = NC * NS
BPW = B // NW
CHUNK = 128
NCHUNK = BPW // CHUNK
NBUF = 3

_mesh = plsc.VectorSubcoreMesh(core_axis_name="c", subcore_axis_name="s")


@functools.partial(
    pl.kernel,
    mesh=_mesh,
    out_type=jax.ShapeDtypeStruct((B, D), jnp.float32),
    scratch_types=[
        pltpu.VMEM((NCHUNK, CHUNK), jnp.int32),
        pltpu.VMEM((CHUNK, D), jnp.float32),
        pltpu.VMEM((CHUNK, D), jnp.float32),
        pltpu.VMEM((CHUNK, D), jnp.float32),
        pltpu.SemaphoreType.DMA,
        pltpu.SemaphoreType.DMA,
    ],
)
def _gather_kernel(t_hbm, w_hbm, out_hbm, idx_v, buf0, buf1, buf2, gsem, wsem):
    wid = lax.axis_index("s") * NC + lax.axis_index("c")
    base = wid * BPW
    pltpu.sync_copy(t_hbm.at[wid], idx_v)

    bufs = (buf0, buf1, buf2)

    def start_gather(c):
        return pltpu.async_copy(w_hbm.at[idx_v.at[c]], bufs[c % NBUF], gsem)

    def start_write(c):
        return pltpu.async_copy(
            bufs[c % NBUF], out_hbm.at[pl.ds(base + c * CHUNK, CHUNK)], wsem
        )

    gathers = [None] * NBUF
    writes = [None] * NBUF
    for c in range(min(NBUF, NCHUNK)):
        gathers[c % NBUF] = start_gather(c)
    for c in range(NCHUNK):
        b = c % NBUF
        gathers[b].wait()
        writes[b] = start_write(c)
        nxt = c + 1
        if NBUF <= nxt < NCHUNK:
            nb = nxt % NBUF
            writes[nb].wait()
            gathers[nb] = start_gather(nxt)
            writes[nb] = None
    for w in writes:
        if w is not None:
            w.wait()


def kernel(t, W):
    t3 = t.reshape(NW, NCHUNK, CHUNK)
    return _gather_kernel(t3, W)
